# Initial kernel scaffold; baseline (speedup 1.0000x reference)
import jax
import jax.numpy as jnp
from jax.experimental import pallas as pl


def kernel(x, W, gamma, beta):
    raise NotImplementedError("write your pallas kernel here")


# speedup vs baseline: 5.1308x; 5.1308x over previous
import functools

import jax
import jax.numpy as jnp
from jax import lax
from jax.experimental import pallas as pl
from jax.experimental.pallas import tpu as pltpu
from jax.experimental.pallas import tpu_sc as plsc

KNN = 20
NEG_INF = float("-inf")


def _knn_body(xr_ref, xf_ref, idx_ref, v_ref):
    b = pl.program_id(0)
    xr = xr_ref[0]
    xf = xf_ref[0]
    dot = lax.dot_general(xr, xf, (((1,), (1,)), ((), ())),
                          preferred_element_type=jnp.float32)
    xx_r = jnp.sum(xr * xr, axis=1, keepdims=True)
    xx_f = jnp.sum(xf * xf, axis=1)[None, :]
    v_ref[...] = (2.0 * dot - xx_r) - xx_f
    R, N = v_ref.shape
    cols = lax.broadcasted_iota(jnp.int32, (R, N), 1)
    k_lane = lax.broadcasted_iota(jnp.int32, (R, KNN), 1)

    def pick(k, acc):
        v = v_ref[...]
        m = jnp.max(v, axis=1, keepdims=True)
        idxk = jnp.min(jnp.where(v == m, cols, N), axis=1, keepdims=True)
        v_ref[...] = jnp.where(cols == idxk, NEG_INF, v)
        return acc + jnp.where(k_lane == k, idxk, 0)

    acc = lax.fori_loop(0, KNN, pick, jnp.zeros((R, KNN), jnp.int32))
    idx_ref[0] = acc + b * N


def _knn_topk(x, tile_rows=256):
    B, N, D = x.shape
    return pl.pallas_call(
        _knn_body,
        grid=(B, N // tile_rows),
        in_specs=[
            pl.BlockSpec((1, tile_rows, D), lambda b, i: (b, i, 0)),
            pl.BlockSpec((1, N, D), lambda b, i: (b, 0, 0)),
        ],
        out_specs=pl.BlockSpec((1, tile_rows, KNN), lambda b, i: (b, i, 0)),
        out_shape=jax.ShapeDtypeStruct((B, N, KNN), jnp.int32),
        scratch_shapes=[pltpu.VMEM((tile_rows, N), jnp.float32)],
    )(x, x)


def _yz_body(x_ref, w_ref, yz_ref):
    yz_ref[...] = lax.dot_general(
        x_ref[...], w_ref[...], (((1,), (1,)), ((), ())),
        preferred_element_type=jnp.float32, precision=lax.Precision.HIGHEST)


def _yz_matmul(x_flat, w_cat, tile_rows=1024):
    BN, D = x_flat.shape
    O2 = w_cat.shape[0]
    return pl.pallas_call(
        _yz_body,
        grid=(BN // tile_rows,),
        in_specs=[
            pl.BlockSpec((tile_rows, D), lambda i: (i, 0)),
            pl.BlockSpec((O2, D), lambda i: (0, 0)),
        ],
        out_specs=pl.BlockSpec((tile_rows, O2), lambda i: (i, 0)),
        out_shape=jax.ShapeDtypeStruct((BN, O2), jnp.float32),
    )(x_flat, w_cat)


def _sc_gather_reduce(y_flat, idx_flat):
    BN, OC = y_flat.shape
    NW = 32
    rows_per_w = BN // NW
    G = 4
    groups = rows_per_w // G
    mesh = plsc.VectorSubcoreMesh(core_axis_name="c", subcore_axis_name="s")

    @functools.partial(
        pl.kernel, mesh=mesh,
        out_type=[
            jax.ShapeDtypeStruct((BN, OC), jnp.float32),
            jax.ShapeDtypeStruct((BN, OC), jnp.float32),
            jax.ShapeDtypeStruct((NW, OC), jnp.float32),
        ],
        scratch_types=[
            pltpu.VMEM((G * KNN,), jnp.int32),
            pltpu.VMEM((G * KNN, 256), jnp.float32),
            pltpu.VMEM((G, 256), jnp.float32),
            pltpu.VMEM((G, 256), jnp.float32),
            pltpu.VMEM((1, 256), jnp.float32),
            pltpu.SemaphoreType.DMA,
        ],
    )
    def sc_kernel(y_hbm, idx_hbm, m_hbm, s_hbm, t2_hbm,
                  idx_v, rows_v, m_v, s_v, t2_v, sem):
        wid = lax.axis_index("s") * 2 + lax.axis_index("c")
        base_n = wid * rows_per_w
        for c in range(OC // 16):
            t2_v[0, pl.ds(c * 16, 16)] = jnp.zeros((16,), jnp.float32)

        def group(g, carry):
            row0 = base_n + g * G
            pltpu.sync_copy(idx_hbm.at[pl.ds(row0 * KNN, G * KNN)], idx_v)
            pltpu.async_copy(y_hbm.at[idx_v], rows_v, sem).wait()

            def per_point(j, carry2):
                for c in range(OC // 16):
                    sl = pl.ds(c * 16, 16)
                    v0 = rows_v[j * KNN, sl]
                    acc_m = v0
                    acc_s = v0
                    acc_q = v0 * v0
                    for kk in range(1, KNN):
                        v = rows_v[j * KNN + kk, sl]
                        acc_m = jnp.maximum(acc_m, v)
                        acc_s = acc_s + v
                        acc_q = acc_q + v * v
                    m_v[j, sl] = acc_m
                    s_v[j, sl] = acc_s
                    t2_v[0, sl] = t2_v[0, sl] + acc_q
                return carry2

            lax.fori_loop(0, G, per_point, 0)
            pltpu.sync_copy(m_v, m_hbm.at[pl.ds(row0, G)])
            pltpu.sync_copy(s_v, s_hbm.at[pl.ds(row0, G)])
            return carry

        lax.fori_loop(0, groups, group, 0)
        pltpu.sync_copy(t2_v, t2_hbm.at[pl.ds(wid, 1)])

    return sc_kernel(y_flat, idx_flat)


def _stats_body(s_ref, z_ref, t2p_ref, t1_ref, z1_ref, z2_ref, zs_ref, t2_ref):
    i = pl.program_id(0)

    @pl.when(i == 0)
    def _init():
        zero = jnp.zeros_like(t1_ref)
        t1_ref[...] = zero
        z1_ref[...] = zero
        z2_ref[...] = zero
        zs_ref[...] = zero
        t2_ref[...] = jnp.sum(t2p_ref[...], axis=0, keepdims=True)

    s = s_ref[...]
    z = z_ref[...]
    t1_ref[...] += jnp.sum(s, axis=0, keepdims=True)
    z1_ref[...] += jnp.sum(z, axis=0, keepdims=True)
    z2_ref[...] += jnp.sum(z * z, axis=0, keepdims=True)
    zs_ref[...] += jnp.sum(z * s, axis=0, keepdims=True)


def _stats(s_arr, z_arr, t2p, tile_rows=1024):
    BN, OC = s_arr.shape
    vec = jax.ShapeDtypeStruct((1, OC), jnp.float32)
    vspec = pl.BlockSpec((1, OC), lambda i: (0, 0))
    return pl.pallas_call(
        _stats_body,
        grid=(BN // tile_rows,),
        in_specs=[
            pl.BlockSpec((tile_rows, OC), lambda i: (i, 0)),
            pl.BlockSpec((tile_rows, OC), lambda i: (i, 0)),
            pl.BlockSpec(t2p.shape, lambda i: (0, 0)),
        ],
        out_specs=[vspec, vspec, vspec, vspec, vspec],
        out_shape=[vec, vec, vec, vec, vec],
    )(s_arr, z_arr, t2p)


def _final_body(z_ref, m_ref, a_ref, bias_ref, o_ref):
    o_ref[...] = jnp.maximum(
        (z_ref[...] + m_ref[...]) * a_ref[...] + bias_ref[...], 0.0)


def _final(z_arr, m_arr, a_vec, bias_vec, tile_rows=1024):
    BN, OC = z_arr.shape
    return pl.pallas_call(
        _final_body,
        grid=(BN // tile_rows,),
        in_specs=[
            pl.BlockSpec((tile_rows, OC), lambda i: (i, 0)),
            pl.BlockSpec((tile_rows, OC), lambda i: (i, 0)),
            pl.BlockSpec((1, OC), lambda i: (0, 0)),
            pl.BlockSpec((1, OC), lambda i: (0, 0)),
        ],
        out_specs=pl.BlockSpec((tile_rows, OC), lambda i: (i, 0)),
        out_shape=jax.ShapeDtypeStruct((BN, OC), jnp.float32),
    )(z_arr, m_arr, a_vec, bias_vec)


def kernel(x, W, gamma, beta):
    B, N, D = x.shape
    OC = W.shape[0]
    BN = B * N

    idx = _knn_topk(x)
    w1 = W[:, :D]
    w_cat = jnp.concatenate([w1, W[:, D:] - w1], axis=0)
    yz = _yz_matmul(x.reshape(BN, D), w_cat)
    y = yz[:, :OC]
    z = yz[:, OC:]

    m_arr, s_arr, t2p = _sc_gather_reduce(y, idx.reshape(BN * KNN))

    t1, z1, z2, zs, t2 = _stats(s_arr, z, t2p)
    cnt = float(BN * KNN)
    mean = (t1 + KNN * z1) / cnt
    e2 = (t2 + 2.0 * zs + KNN * z2) / cnt
    var = e2 - mean * mean
    inv = lax.rsqrt(var + 1e-5)
    a_vec = gamma[None, :] * inv
    bias_vec = beta[None, :] - mean * a_vec

    out = _final(z, m_arr, a_vec, bias_vec)
    return out.reshape(B, N, OC)


# speedup vs baseline: 6.0273x; 1.1747x over previous
import functools

import jax
import jax.numpy as jnp
from jax import lax
from jax.experimental import pallas as pl
from jax.experimental.pallas import tpu as pltpu
from jax.experimental.pallas import tpu_sc as plsc

KNN = 20
NEG_INF = float("-inf")


def _knn_body(xr_ref, xf_ref, idx_ref, v_ref):
    b = pl.program_id(0)
    xr = xr_ref[0]
    xf = xf_ref[0]
    dot = lax.dot_general(xr, xf, (((1,), (1,)), ((), ())),
                          preferred_element_type=jnp.float32)
    xx_r = jnp.sum(xr * xr, axis=1, keepdims=True)
    xx_f = jnp.sum(xf * xf, axis=1)[None, :]
    v_ref[...] = (2.0 * dot - xx_r) - xx_f
    R, N = v_ref.shape
    cols = lax.broadcasted_iota(jnp.int32, (R, N), 1)
    k_lane = lax.broadcasted_iota(jnp.int32, (R, KNN), 1)

    def pick(k, acc):
        v = v_ref[...]
        m = jnp.max(v, axis=1, keepdims=True)
        sel = v == m
        idxk = jnp.min(jnp.where(sel, cols, N), axis=1, keepdims=True)
        v_ref[...] = jnp.where(sel, NEG_INF, v)
        return acc + jnp.where(k_lane == k, idxk, 0)

    acc = lax.fori_loop(0, KNN, pick, jnp.zeros((R, KNN), jnp.int32))
    idx_ref[0] = acc + b * N


def _knn_topk(x, tile_rows=256):
    B, N, D = x.shape
    return pl.pallas_call(
        _knn_body,
        grid=(B, N // tile_rows),
        in_specs=[
            pl.BlockSpec((1, tile_rows, D), lambda b, i: (b, i, 0)),
            pl.BlockSpec((1, N, D), lambda b, i: (b, 0, 0)),
        ],
        out_specs=pl.BlockSpec((1, tile_rows, KNN), lambda b, i: (b, i, 0)),
        out_shape=jax.ShapeDtypeStruct((B, N, KNN), jnp.int32),
        scratch_shapes=[pltpu.VMEM((tile_rows, N), jnp.float32)],
    )(x, x)


def _yz_body(x_ref, w_ref, yz_ref):
    yz_ref[...] = lax.dot_general(
        x_ref[...], w_ref[...], (((1,), (1,)), ((), ())),
        preferred_element_type=jnp.float32, precision=lax.Precision.HIGHEST)


def _yz_matmul(x_flat, w_cat, tile_rows=1024):
    BN, D = x_flat.shape
    O2 = w_cat.shape[0]
    return pl.pallas_call(
        _yz_body,
        grid=(BN // tile_rows,),
        in_specs=[
            pl.BlockSpec((tile_rows, D), lambda i: (i, 0)),
            pl.BlockSpec((O2, D), lambda i: (0, 0)),
        ],
        out_specs=pl.BlockSpec((tile_rows, O2), lambda i: (i, 0)),
        out_shape=jax.ShapeDtypeStruct((BN, O2), jnp.float32),
    )(x_flat, w_cat)


def _sc_gather_reduce(y_flat, idx_flat):
    BN, OC = y_flat.shape
    NW = 32
    rows_per_w = BN // NW
    G = 4
    GK = G * KNN
    groups = rows_per_w // G
    mesh = plsc.VectorSubcoreMesh(core_axis_name="c", subcore_axis_name="s")

    @functools.partial(
        pl.kernel, mesh=mesh,
        out_type=[
            jax.ShapeDtypeStruct((BN, OC), jnp.float32),
            jax.ShapeDtypeStruct((BN, OC), jnp.float32),
            jax.ShapeDtypeStruct((NW, OC), jnp.float32),
        ],
        scratch_types=[
            pltpu.VMEM((2, GK), jnp.int32),
            pltpu.VMEM((2, GK, 256), jnp.float32),
            pltpu.VMEM((2, G, 256), jnp.float32),
            pltpu.VMEM((2, G, 256), jnp.float32),
            pltpu.VMEM((1, 256), jnp.float32),
            pltpu.SemaphoreType.DMA,
            pltpu.SemaphoreType.DMA,
            pltpu.SemaphoreType.DMA,
            pltpu.SemaphoreType.DMA,
            pltpu.SemaphoreType.DMA,
            pltpu.SemaphoreType.DMA,
        ],
    )
    def sc_kernel(y_hbm, idx_hbm, m_hbm, s_hbm, t2_hbm,
                  idx_v, rows_v, m_v, s_v, t2_v,
                  si0, si1, sr0, sr1, sw0, sw1):
        sem_i = (si0, si1)
        sem_r = (sr0, sr1)
        sem_w = (sw0, sw1)
        wid = lax.axis_index("s") * 2 + lax.axis_index("c")
        base_n = wid * rows_per_w
        for c in range(OC // 16):
            t2_v[0, pl.ds(c * 16, 16)] = jnp.zeros((16,), jnp.float32)

        def idx_copy(g, b):
            return pltpu.make_async_copy(
                idx_hbm.at[pl.ds((base_n + g * G) * KNN, GK)],
                idx_v.at[b], sem_i[b])

        def rows_copy(b):
            return pltpu.make_async_copy(y_hbm.at[idx_v.at[b]],
                                         rows_v.at[b], sem_r[b])

        def write_copies(g, b):
            row0 = base_n + g * G
            return (
                pltpu.make_async_copy(m_v.at[b], m_hbm.at[pl.ds(row0, G)],
                                      sem_w[b]),
                pltpu.make_async_copy(s_v.at[b], s_hbm.at[pl.ds(row0, G)],
                                      sem_w[b]),
            )

        def compute(b):
            def per_point(j, carry2):
                for c in range(OC // 16):
                    sl = pl.ds(c * 16, 16)
                    v0 = rows_v[b, j * KNN, sl]
                    acc_m = v0
                    acc_s = v0
                    acc_q = v0 * v0
                    for kk in range(1, KNN):
                        v = rows_v[b, j * KNN + kk, sl]
                        acc_m = jnp.maximum(acc_m, v)
                        acc_s = acc_s + v
                        acc_q = acc_q + v * v
                    m_v[b, j, sl] = acc_m
                    s_v[b, j, sl] = acc_s
                    t2_v[0, sl] = t2_v[0, sl] + acc_q
                return carry2

            lax.fori_loop(0, G, per_point, 0)

        idx_copy(0, 0).start()
        idx_copy(0, 0).wait()
        rows_copy(0).start()
        idx_copy(1, 1).start()

        def steady(g, b):
            rows_copy(b).wait()
            idx_copy(g + 1, 1 - b).wait()
            rows_copy(1 - b).start()
            idx_copy(g + 2, b).start()

            @pl.when(g >= 2)
            def _drain():
                wm, ws = write_copies(g - 2, b)
                wm.wait()
                ws.wait()

            compute(b)
            wm, ws = write_copies(g, b)
            wm.start()
            ws.start()

        def pair(p, carry):
            steady(2 * p, 0)
            steady(2 * p + 1, 1)
            return carry

        lax.fori_loop(0, groups // 2 - 1, pair, 0)

        g0 = groups - 2
        rows_copy(0).wait()
        idx_copy(g0 + 1, 1).wait()
        rows_copy(1).start()
        wm, ws = write_copies(g0 - 2, 0)
        wm.wait()
        ws.wait()
        compute(0)
        wm, ws = write_copies(g0, 0)
        wm.start()
        ws.start()

        g1 = groups - 1
        rows_copy(1).wait()
        wm, ws = write_copies(g1 - 2, 1)
        wm.wait()
        ws.wait()
        compute(1)
        wm, ws = write_copies(g1, 1)
        wm.start()
        ws.start()

        wm, ws = write_copies(g0, 0)
        wm.wait()
        ws.wait()
        wm, ws = write_copies(g1, 1)
        wm.wait()
        ws.wait()
        pltpu.sync_copy(t2_v, t2_hbm.at[pl.ds(wid, 1)])

    return sc_kernel(y_flat, idx_flat)


def _stats_body(s_ref, z_ref, t2p_ref, t1_ref, z1_ref, z2_ref, zs_ref, t2_ref):
    i = pl.program_id(0)

    @pl.when(i == 0)
    def _init():
        zero = jnp.zeros_like(t1_ref)
        t1_ref[...] = zero
        z1_ref[...] = zero
        z2_ref[...] = zero
        zs_ref[...] = zero
        t2_ref[...] = jnp.sum(t2p_ref[...], axis=0, keepdims=True)

    s = s_ref[...]
    z = z_ref[...]
    t1_ref[...] += jnp.sum(s, axis=0, keepdims=True)
    z1_ref[...] += jnp.sum(z, axis=0, keepdims=True)
    z2_ref[...] += jnp.sum(z * z, axis=0, keepdims=True)
    zs_ref[...] += jnp.sum(z * s, axis=0, keepdims=True)


def _stats(s_arr, z_arr, t2p, tile_rows=1024):
    BN, OC = s_arr.shape
    vec = jax.ShapeDtypeStruct((1, OC), jnp.float32)
    vspec = pl.BlockSpec((1, OC), lambda i: (0, 0))
    return pl.pallas_call(
        _stats_body,
        grid=(BN // tile_rows,),
        in_specs=[
            pl.BlockSpec((tile_rows, OC), lambda i: (i, 0)),
            pl.BlockSpec((tile_rows, OC), lambda i: (i, 0)),
            pl.BlockSpec(t2p.shape, lambda i: (0, 0)),
        ],
        out_specs=[vspec, vspec, vspec, vspec, vspec],
        out_shape=[vec, vec, vec, vec, vec],
    )(s_arr, z_arr, t2p)


def _final_body(z_ref, m_ref, a_ref, bias_ref, o_ref):
    o_ref[...] = jnp.maximum(
        (z_ref[...] + m_ref[...]) * a_ref[...] + bias_ref[...], 0.0)


def _final(z_arr, m_arr, a_vec, bias_vec, tile_rows=1024):
    BN, OC = z_arr.shape
    return pl.pallas_call(
        _final_body,
        grid=(BN // tile_rows,),
        in_specs=[
            pl.BlockSpec((tile_rows, OC), lambda i: (i, 0)),
            pl.BlockSpec((tile_rows, OC), lambda i: (i, 0)),
            pl.BlockSpec((1, OC), lambda i: (0, 0)),
            pl.BlockSpec((1, OC), lambda i: (0, 0)),
        ],
        out_specs=pl.BlockSpec((tile_rows, OC), lambda i: (i, 0)),
        out_shape=jax.ShapeDtypeStruct((BN, OC), jnp.float32),
    )(z_arr, m_arr, a_vec, bias_vec)


def kernel(x, W, gamma, beta):
    B, N, D = x.shape
    OC = W.shape[0]
    BN = B * N

    idx = _knn_topk(x)
    w1 = W[:, :D]
    w_cat = jnp.concatenate([w1, W[:, D:] - w1], axis=0)
    yz = _yz_matmul(x.reshape(BN, D), w_cat)
    y = yz[:, :OC]
    z = yz[:, OC:]

    m_arr, s_arr, t2p = _sc_gather_reduce(y, idx.reshape(BN * KNN))

    t1, z1, z2, zs, t2 = _stats(s_arr, z, t2p)
    cnt = float(BN * KNN)
    mean = (t1 + KNN * z1) / cnt
    e2 = (t2 + 2.0 * zs + KNN * z2) / cnt
    var = e2 - mean * mean
    inv = lax.rsqrt(var + 1e-5)
    a_vec = gamma[None, :] * inv
    bias_vec = beta[None, :] - mean * a_vec

    out = _final(z, m_arr, a_vec, bias_vec)
    return out.reshape(B, N, OC)


# speedup vs baseline: 8.3905x; 1.3921x over previous
import functools

import jax
import jax.numpy as jnp
from jax import lax
from jax.experimental import pallas as pl
from jax.experimental.pallas import tpu as pltpu
from jax.experimental.pallas import tpu_sc as plsc

KNN = 20
NEG_INF = float("-inf")


def _knn_body(base, xr_ref, xf_ref, nc_ref, idx_ref, v_ref):
    xr = xr_ref[0]
    xf = xf_ref[0]
    dot = lax.dot_general(xr, xf, (((1,), (1,)), ((), ())),
                          preferred_element_type=jnp.float32)
    xx_r = jnp.sum(xr * xr, axis=1, keepdims=True)
    xx_f = jnp.sum(xf * xf, axis=1)[None, :]
    v_ref[...] = (2.0 * dot - xx_r) - xx_f
    R, N = v_ref.shape
    ncols = nc_ref[...]
    k_lane = lax.broadcasted_iota(jnp.int32, (R, KNN), 1)

    def pick(k, acc):
        v = v_ref[...]
        m = jnp.max(v, axis=1, keepdims=True)
        sel = v == m
        idxf = jnp.max(jnp.where(sel, ncols, NEG_INF), axis=1, keepdims=True)
        idxk = (-idxf).astype(jnp.int32)
        v_ref[...] = jnp.where(sel, NEG_INF, v)
        return acc + jnp.where(k_lane == k, idxk, 0)

    acc = lax.fori_loop(0, KNN, pick, jnp.zeros((R, KNN), jnp.int32))
    idx_ref[...] = acc + base


def _knn_topk_batch(x, ncols, b, tile_rows=256):
    B, N, D = x.shape
    return pl.pallas_call(
        functools.partial(_knn_body, b * N),
        grid=(N // tile_rows,),
        in_specs=[
            pl.BlockSpec((1, tile_rows, D), lambda i, _b=b: (_b, i, 0)),
            pl.BlockSpec((1, N, D), lambda i, _b=b: (_b, 0, 0)),
            pl.BlockSpec((1, N), lambda i: (0, 0)),
        ],
        out_specs=pl.BlockSpec((tile_rows, KNN), lambda i: (i, 0)),
        out_shape=jax.ShapeDtypeStruct((N, KNN), jnp.int32),
        scratch_shapes=[pltpu.VMEM((tile_rows, N), jnp.float32)],
    )(x, x, ncols)


def _yz_body(x_ref, w_ref, yz_ref):
    yz_ref[...] = lax.dot_general(
        x_ref[...], w_ref[...], (((1,), (1,)), ((), ())),
        preferred_element_type=jnp.float32, precision=lax.Precision.HIGHEST)


def _yz_matmul(x_flat, w_cat, tile_rows=1024):
    BN, D = x_flat.shape
    O2 = w_cat.shape[0]
    return pl.pallas_call(
        _yz_body,
        grid=(BN // tile_rows,),
        in_specs=[
            pl.BlockSpec((tile_rows, D), lambda i: (i, 0)),
            pl.BlockSpec((O2, D), lambda i: (0, 0)),
        ],
        out_specs=pl.BlockSpec((tile_rows, O2), lambda i: (i, 0)),
        out_shape=jax.ShapeDtypeStruct((BN, O2), jnp.float32),
    )(x_flat, w_cat)


def _sc_gather_reduce(y_flat, idx_flat):
    _, OC = y_flat.shape
    P = idx_flat.shape[0] // KNN
    NW = 32
    rows_per_w = P // NW
    G = 4
    GK = G * KNN
    groups = rows_per_w // G
    mesh = plsc.VectorSubcoreMesh(core_axis_name="c", subcore_axis_name="s")

    @functools.partial(
        pl.kernel, mesh=mesh,
        out_type=[
            jax.ShapeDtypeStruct((P, OC), jnp.float32),
            jax.ShapeDtypeStruct((P, OC), jnp.float32),
            jax.ShapeDtypeStruct((NW, OC), jnp.float32),
        ],
        scratch_types=[
            pltpu.VMEM((2, GK), jnp.int32),
            pltpu.VMEM((2, GK, 256), jnp.float32),
            pltpu.VMEM((2, G, 256), jnp.float32),
            pltpu.VMEM((2, G, 256), jnp.float32),
            pltpu.VMEM((1, 256), jnp.float32),
            pltpu.SemaphoreType.DMA,
            pltpu.SemaphoreType.DMA,
            pltpu.SemaphoreType.DMA,
            pltpu.SemaphoreType.DMA,
            pltpu.SemaphoreType.DMA,
            pltpu.SemaphoreType.DMA,
        ],
    )
    def sc_kernel(y_hbm, idx_hbm, m_hbm, s_hbm, t2_hbm,
                  idx_v, rows_v, m_v, s_v, t2_v,
                  si0, si1, sr0, sr1, sw0, sw1):
        sem_i = (si0, si1)
        sem_r = (sr0, sr1)
        sem_w = (sw0, sw1)
        wid = lax.axis_index("s") * 2 + lax.axis_index("c")
        base_n = wid * rows_per_w
        for c in range(OC // 16):
            t2_v[0, pl.ds(c * 16, 16)] = jnp.zeros((16,), jnp.float32)

        def idx_copy(g, b):
            return pltpu.make_async_copy(
                idx_hbm.at[pl.ds((base_n + g * G) * KNN, GK)],
                idx_v.at[b], sem_i[b])

        def rows_copy(b):
            return pltpu.make_async_copy(y_hbm.at[idx_v.at[b]],
                                         rows_v.at[b], sem_r[b])

        def write_copies(g, b):
            row0 = base_n + g * G
            return (
                pltpu.make_async_copy(m_v.at[b], m_hbm.at[pl.ds(row0, G)],
                                      sem_w[b]),
                pltpu.make_async_copy(s_v.at[b], s_hbm.at[pl.ds(row0, G)],
                                      sem_w[b]),
            )

        def compute(b):
            def per_point(j, carry2):
                for c in range(OC // 16):
                    sl = pl.ds(c * 16, 16)
                    v0 = rows_v[b, j * KNN, sl]
                    acc_m = v0
                    acc_s = v0
                    acc_q = v0 * v0
                    for kk in range(1, KNN):
                        v = rows_v[b, j * KNN + kk, sl]
                        acc_m = jnp.maximum(acc_m, v)
                        acc_s = acc_s + v
                        acc_q = acc_q + v * v
                    m_v[b, j, sl] = acc_m
                    s_v[b, j, sl] = acc_s
                    t2_v[0, sl] = t2_v[0, sl] + acc_q
                return carry2

            lax.fori_loop(0, G, per_point, 0)

        idx_copy(0, 0).start()
        idx_copy(0, 0).wait()
        rows_copy(0).start()
        idx_copy(1, 1).start()

        def steady(g, b):
            rows_copy(b).wait()
            idx_copy(g + 1, 1 - b).wait()
            rows_copy(1 - b).start()
            idx_copy(g + 2, b).start()

            @pl.when(g >= 2)
            def _drain():
                wm, ws = write_copies(g - 2, b)
                wm.wait()
                ws.wait()

            compute(b)
            wm, ws = write_copies(g, b)
            wm.start()
            ws.start()

        def pair(p, carry):
            steady(2 * p, 0)
            steady(2 * p + 1, 1)
            return carry

        lax.fori_loop(0, groups // 2 - 1, pair, 0)

        g0 = groups - 2
        rows_copy(0).wait()
        idx_copy(g0 + 1, 1).wait()
        rows_copy(1).start()
        wm, ws = write_copies(g0 - 2, 0)
        wm.wait()
        ws.wait()
        compute(0)
        wm, ws = write_copies(g0, 0)
        wm.start()
        ws.start()

        g1 = groups - 1
        rows_copy(1).wait()
        wm, ws = write_copies(g1 - 2, 1)
        wm.wait()
        ws.wait()
        compute(1)
        wm, ws = write_copies(g1, 1)
        wm.start()
        ws.start()

        wm, ws = write_copies(g0, 0)
        wm.wait()
        ws.wait()
        wm, ws = write_copies(g1, 1)
        wm.wait()
        ws.wait()
        pltpu.sync_copy(t2_v, t2_hbm.at[pl.ds(wid, 1)])

    return sc_kernel(y_flat, idx_flat)


def _stats_body(s_ref, z_ref, t2p_ref, t1_ref, z1_ref, z2_ref, zs_ref, t2_ref):
    i = pl.program_id(0)

    @pl.when(i == 0)
    def _init():
        zero = jnp.zeros_like(t1_ref)
        t1_ref[...] = zero
        z1_ref[...] = zero
        z2_ref[...] = zero
        zs_ref[...] = zero
        t2_ref[...] = jnp.sum(t2p_ref[...], axis=0, keepdims=True)

    s = s_ref[...]
    z = z_ref[...]
    t1_ref[...] += jnp.sum(s, axis=0, keepdims=True)
    z1_ref[...] += jnp.sum(z, axis=0, keepdims=True)
    z2_ref[...] += jnp.sum(z * z, axis=0, keepdims=True)
    zs_ref[...] += jnp.sum(z * s, axis=0, keepdims=True)


def _stats(s_arr, z_arr, t2p, tile_rows=1024):
    BN, OC = s_arr.shape
    vec = jax.ShapeDtypeStruct((1, OC), jnp.float32)
    vspec = pl.BlockSpec((1, OC), lambda i: (0, 0))
    return pl.pallas_call(
        _stats_body,
        grid=(BN // tile_rows,),
        in_specs=[
            pl.BlockSpec((tile_rows, OC), lambda i: (i, 0)),
            pl.BlockSpec((tile_rows, OC), lambda i: (i, 0)),
            pl.BlockSpec(t2p.shape, lambda i: (0, 0)),
        ],
        out_specs=[vspec, vspec, vspec, vspec, vspec],
        out_shape=[vec, vec, vec, vec, vec],
    )(s_arr, z_arr, t2p)


def _final_body(z_ref, m_ref, a_ref, bias_ref, o_ref):
    o_ref[...] = jnp.maximum(
        (z_ref[...] + m_ref[...]) * a_ref[...] + bias_ref[...], 0.0)


def _final(z_arr, m_arr, a_vec, bias_vec, tile_rows=1024):
    BN, OC = z_arr.shape
    return pl.pallas_call(
        _final_body,
        grid=(BN // tile_rows,),
        in_specs=[
            pl.BlockSpec((tile_rows, OC), lambda i: (i, 0)),
            pl.BlockSpec((tile_rows, OC), lambda i: (i, 0)),
            pl.BlockSpec((1, OC), lambda i: (0, 0)),
            pl.BlockSpec((1, OC), lambda i: (0, 0)),
        ],
        out_specs=pl.BlockSpec((tile_rows, OC), lambda i: (i, 0)),
        out_shape=jax.ShapeDtypeStruct((BN, OC), jnp.float32),
    )(z_arr, m_arr, a_vec, bias_vec)


def kernel(x, W, gamma, beta):
    B, N, D = x.shape
    OC = W.shape[0]
    BN = B * N

    w1 = W[:, :D]
    w_cat = jnp.concatenate([w1, W[:, D:] - w1], axis=0)
    yz = _yz_matmul(x.reshape(BN, D), w_cat)
    y = yz[:, :OC]
    z = yz[:, OC:]

    ncols = -lax.iota(jnp.float32, N)[None, :]
    m_parts, s_parts, t2_parts = [], [], []
    for b in range(B):
        idx_b = _knn_topk_batch(x, ncols, b)
        m_b, s_b, t2p_b = _sc_gather_reduce(y, idx_b.reshape(N * KNN))
        m_parts.append(m_b)
        s_parts.append(s_b)
        t2_parts.append(t2p_b)
    m_arr = jnp.concatenate(m_parts, axis=0)
    s_arr = jnp.concatenate(s_parts, axis=0)
    t2p = jnp.concatenate(t2_parts, axis=0)

    t1, z1, z2, zs, t2 = _stats(s_arr, z, t2p)
    cnt = float(BN * KNN)
    mean = (t1 + KNN * z1) / cnt
    e2 = (t2 + 2.0 * zs + KNN * z2) / cnt
    var = e2 - mean * mean
    inv = lax.rsqrt(var + 1e-5)
    a_vec = gamma[None, :] * inv
    bias_vec = beta[None, :] - mean * a_vec

    out = _final(z, m_arr, a_vec, bias_vec)
    return out.reshape(B, N, OC)


# speedup vs baseline: 8.3912x; 1.0001x over previous
import functools

import jax
import jax.numpy as jnp
from jax import lax
from jax.experimental import pallas as pl
from jax.experimental.pallas import tpu as pltpu
from jax.experimental.pallas import tpu_sc as plsc

KNN = 20
NEG_INF = float("-inf")


def _knn_body(base, xr_ref, xf_ref, nc_ref, idx_ref, v_ref):
    xr = xr_ref[0]
    xf = xf_ref[0]
    dot = lax.dot_general(xr, xf, (((1,), (1,)), ((), ())),
                          preferred_element_type=jnp.float32)
    xx_r = jnp.sum(xr * xr, axis=1, keepdims=True)
    xx_f = jnp.sum(xf * xf, axis=1)[None, :]
    v_ref[...] = (2.0 * dot - xx_r) - xx_f
    R, N = v_ref.shape
    ncols = nc_ref[...]
    k_lane = lax.broadcasted_iota(jnp.int32, (R, KNN), 1)

    def pick2(k, acc):
        v = v_ref[...]
        m1 = jnp.max(v, axis=1, keepdims=True)
        sel1 = v == m1
        idx1f = jnp.max(jnp.where(sel1, ncols, NEG_INF), axis=1, keepdims=True)
        v2 = jnp.where(sel1, NEG_INF, v)
        m2 = jnp.max(v2, axis=1, keepdims=True)
        sel2 = v2 == m2
        idx2f = jnp.max(jnp.where(sel2, ncols, NEG_INF), axis=1, keepdims=True)
        v_ref[...] = jnp.where(sel2, NEG_INF, v2)
        idx1 = (-idx1f).astype(jnp.int32)
        idx2 = (-idx2f).astype(jnp.int32)
        return (acc + jnp.where(k_lane == 2 * k, idx1, 0)
                + jnp.where(k_lane == 2 * k + 1, idx2, 0))

    acc = lax.fori_loop(0, KNN // 2, pick2, jnp.zeros((R, KNN), jnp.int32))
    idx_ref[...] = acc + base


def _knn_topk_batch(x, ncols, b, tile_rows=256):
    B, N, D = x.shape
    return pl.pallas_call(
        functools.partial(_knn_body, b * N),
        grid=(N // tile_rows,),
        in_specs=[
            pl.BlockSpec((1, tile_rows, D), lambda i, _b=b: (_b, i, 0)),
            pl.BlockSpec((1, N, D), lambda i, _b=b: (_b, 0, 0)),
            pl.BlockSpec((1, N), lambda i: (0, 0)),
        ],
        out_specs=pl.BlockSpec((tile_rows, KNN), lambda i: (i, 0)),
        out_shape=jax.ShapeDtypeStruct((N, KNN), jnp.int32),
        scratch_shapes=[pltpu.VMEM((tile_rows, N), jnp.float32)],
    )(x, x, ncols)


def _yz_body(x_ref, w_ref, yz_ref):
    yz_ref[...] = lax.dot_general(
        x_ref[...], w_ref[...], (((1,), (1,)), ((), ())),
        preferred_element_type=jnp.float32, precision=lax.Precision.HIGHEST)


def _yz_matmul(x_flat, w_cat, tile_rows=1024):
    BN, D = x_flat.shape
    O2 = w_cat.shape[0]
    return pl.pallas_call(
        _yz_body,
        grid=(BN // tile_rows,),
        in_specs=[
            pl.BlockSpec((tile_rows, D), lambda i: (i, 0)),
            pl.BlockSpec((O2, D), lambda i: (0, 0)),
        ],
        out_specs=pl.BlockSpec((tile_rows, O2), lambda i: (i, 0)),
        out_shape=jax.ShapeDtypeStruct((BN, O2), jnp.float32),
    )(x_flat, w_cat)


def _sc_gather_reduce(y_flat, idx_flat):
    _, OC = y_flat.shape
    P = idx_flat.shape[0] // KNN
    NW = 32
    rows_per_w = P // NW
    G = 4
    GK = G * KNN
    groups = rows_per_w // G
    mesh = plsc.VectorSubcoreMesh(core_axis_name="c", subcore_axis_name="s")

    @functools.partial(
        pl.kernel, mesh=mesh,
        out_type=[
            jax.ShapeDtypeStruct((P, OC), jnp.float32),
            jax.ShapeDtypeStruct((P, OC), jnp.float32),
            jax.ShapeDtypeStruct((NW, OC), jnp.float32),
        ],
        scratch_types=[
            pltpu.VMEM((2, GK), jnp.int32),
            pltpu.VMEM((2, GK, 256), jnp.float32),
            pltpu.VMEM((2, G, 256), jnp.float32),
            pltpu.VMEM((2, G, 256), jnp.float32),
            pltpu.VMEM((1, 256), jnp.float32),
            pltpu.SemaphoreType.DMA,
            pltpu.SemaphoreType.DMA,
            pltpu.SemaphoreType.DMA,
            pltpu.SemaphoreType.DMA,
            pltpu.SemaphoreType.DMA,
            pltpu.SemaphoreType.DMA,
        ],
    )
    def sc_kernel(y_hbm, idx_hbm, m_hbm, s_hbm, t2_hbm,
                  idx_v, rows_v, m_v, s_v, t2_v,
                  si0, si1, sr0, sr1, sw0, sw1):
        sem_i = (si0, si1)
        sem_r = (sr0, sr1)
        sem_w = (sw0, sw1)
        wid = lax.axis_index("s") * 2 + lax.axis_index("c")
        base_n = wid * rows_per_w
        for c in range(OC // 16):
            t2_v[0, pl.ds(c * 16, 16)] = jnp.zeros((16,), jnp.float32)

        def idx_copy(g, b):
            return pltpu.make_async_copy(
                idx_hbm.at[pl.ds((base_n + g * G) * KNN, GK)],
                idx_v.at[b], sem_i[b])

        def rows_copy(b):
            return pltpu.make_async_copy(y_hbm.at[idx_v.at[b]],
                                         rows_v.at[b], sem_r[b])

        def write_copies(g, b):
            row0 = base_n + g * G
            return (
                pltpu.make_async_copy(m_v.at[b], m_hbm.at[pl.ds(row0, G)],
                                      sem_w[b]),
                pltpu.make_async_copy(s_v.at[b], s_hbm.at[pl.ds(row0, G)],
                                      sem_w[b]),
            )

        def compute(b):
            def per_point(j, carry2):
                for c in range(OC // 16):
                    sl = pl.ds(c * 16, 16)
                    v0 = rows_v[b, j * KNN, sl]
                    acc_m = v0
                    acc_s = v0
                    acc_q = v0 * v0
                    for kk in range(1, KNN):
                        v = rows_v[b, j * KNN + kk, sl]
                        acc_m = jnp.maximum(acc_m, v)
                        acc_s = acc_s + v
                        acc_q = acc_q + v * v
                    m_v[b, j, sl] = acc_m
                    s_v[b, j, sl] = acc_s
                    t2_v[0, sl] = t2_v[0, sl] + acc_q
                return carry2

            lax.fori_loop(0, G, per_point, 0)

        idx_copy(0, 0).start()
        idx_copy(0, 0).wait()
        rows_copy(0).start()
        idx_copy(1, 1).start()

        def steady(g, b):
            rows_copy(b).wait()
            idx_copy(g + 1, 1 - b).wait()
            rows_copy(1 - b).start()
            idx_copy(g + 2, b).start()

            @pl.when(g >= 2)
            def _drain():
                wm, ws = write_copies(g - 2, b)
                wm.wait()
                ws.wait()

            compute(b)
            wm, ws = write_copies(g, b)
            wm.start()
            ws.start()

        def pair(p, carry):
            steady(2 * p, 0)
            steady(2 * p + 1, 1)
            return carry

        lax.fori_loop(0, groups // 2 - 1, pair, 0)

        g0 = groups - 2
        rows_copy(0).wait()
        idx_copy(g0 + 1, 1).wait()
        rows_copy(1).start()
        wm, ws = write_copies(g0 - 2, 0)
        wm.wait()
        ws.wait()
        compute(0)
        wm, ws = write_copies(g0, 0)
        wm.start()
        ws.start()

        g1 = groups - 1
        rows_copy(1).wait()
        wm, ws = write_copies(g1 - 2, 1)
        wm.wait()
        ws.wait()
        compute(1)
        wm, ws = write_copies(g1, 1)
        wm.start()
        ws.start()

        wm, ws = write_copies(g0, 0)
        wm.wait()
        ws.wait()
        wm, ws = write_copies(g1, 1)
        wm.wait()
        ws.wait()
        pltpu.sync_copy(t2_v, t2_hbm.at[pl.ds(wid, 1)])

    return sc_kernel(y_flat, idx_flat)


def _stats_body(s_ref, z_ref, t2p_ref, t1_ref, z1_ref, z2_ref, zs_ref, t2_ref):
    i = pl.program_id(0)

    @pl.when(i == 0)
    def _init():
        zero = jnp.zeros_like(t1_ref)
        t1_ref[...] = zero
        z1_ref[...] = zero
        z2_ref[...] = zero
        zs_ref[...] = zero
        t2_ref[...] = jnp.sum(t2p_ref[...], axis=0, keepdims=True)

    s = s_ref[...]
    z = z_ref[...]
    t1_ref[...] += jnp.sum(s, axis=0, keepdims=True)
    z1_ref[...] += jnp.sum(z, axis=0, keepdims=True)
    z2_ref[...] += jnp.sum(z * z, axis=0, keepdims=True)
    zs_ref[...] += jnp.sum(z * s, axis=0, keepdims=True)


def _stats(s_arr, z_arr, t2p, tile_rows=1024):
    BN, OC = s_arr.shape
    vec = jax.ShapeDtypeStruct((1, OC), jnp.float32)
    vspec = pl.BlockSpec((1, OC), lambda i: (0, 0))
    return pl.pallas_call(
        _stats_body,
        grid=(BN // tile_rows,),
        in_specs=[
            pl.BlockSpec((tile_rows, OC), lambda i: (i, 0)),
            pl.BlockSpec((tile_rows, OC), lambda i: (i, 0)),
            pl.BlockSpec(t2p.shape, lambda i: (0, 0)),
        ],
        out_specs=[vspec, vspec, vspec, vspec, vspec],
        out_shape=[vec, vec, vec, vec, vec],
    )(s_arr, z_arr, t2p)


def _final_body(z_ref, m_ref, a_ref, bias_ref, o_ref):
    o_ref[...] = jnp.maximum(
        (z_ref[...] + m_ref[...]) * a_ref[...] + bias_ref[...], 0.0)


def _final(z_arr, m_arr, a_vec, bias_vec, tile_rows=1024):
    BN, OC = z_arr.shape
    return pl.pallas_call(
        _final_body,
        grid=(BN // tile_rows,),
        in_specs=[
            pl.BlockSpec((tile_rows, OC), lambda i: (i, 0)),
            pl.BlockSpec((tile_rows, OC), lambda i: (i, 0)),
            pl.BlockSpec((1, OC), lambda i: (0, 0)),
            pl.BlockSpec((1, OC), lambda i: (0, 0)),
        ],
        out_specs=pl.BlockSpec((tile_rows, OC), lambda i: (i, 0)),
        out_shape=jax.ShapeDtypeStruct((BN, OC), jnp.float32),
    )(z_arr, m_arr, a_vec, bias_vec)


def kernel(x, W, gamma, beta):
    B, N, D = x.shape
    OC = W.shape[0]
    BN = B * N

    w1 = W[:, :D]
    w_cat = jnp.concatenate([w1, W[:, D:] - w1], axis=0)
    yz = _yz_matmul(x.reshape(BN, D), w_cat)
    y = yz[:, :OC]
    z = yz[:, OC:]

    ncols = -lax.iota(jnp.float32, N)[None, :]
    m_parts, s_parts, t2_parts = [], [], []
    for b in range(B):
        idx_b = _knn_topk_batch(x, ncols, b)
        m_b, s_b, t2p_b = _sc_gather_reduce(y, idx_b.reshape(N * KNN))
        m_parts.append(m_b)
        s_parts.append(s_b)
        t2_parts.append(t2p_b)
    m_arr = jnp.concatenate(m_parts, axis=0)
    s_arr = jnp.concatenate(s_parts, axis=0)
    t2p = jnp.concatenate(t2_parts, axis=0)

    t1, z1, z2, zs, t2 = _stats(s_arr, z, t2p)
    cnt = float(BN * KNN)
    mean = (t1 + KNN * z1) / cnt
    e2 = (t2 + 2.0 * zs + KNN * z2) / cnt
    var = e2 - mean * mean
    inv = lax.rsqrt(var + 1e-5)
    a_vec = gamma[None, :] * inv
    bias_vec = beta[None, :] - mean * a_vec

    out = _final(z, m_arr, a_vec, bias_vec)
    return out.reshape(B, N, OC)


# speedup vs baseline: 8.6145x; 1.0266x over previous
import functools

import jax
import jax.numpy as jnp
from jax import lax
from jax.experimental import pallas as pl
from jax.experimental.pallas import tpu as pltpu
from jax.experimental.pallas import tpu_sc as plsc

KNN = 20
NEG_INF = float("-inf")


def _knn_body(base, xr_ref, xf_ref, nc_ref, idx_ref, v_ref):
    xr = xr_ref[0]
    xf = xf_ref[0]
    dot = lax.dot_general(xr, xf, (((1,), (1,)), ((), ())),
                          preferred_element_type=jnp.float32)
    xx_r = jnp.sum(xr * xr, axis=1, keepdims=True)
    xx_f = jnp.sum(xf * xf, axis=1)[None, :]
    v_ref[...] = (2.0 * dot - xx_r) - xx_f
    R, N = v_ref.shape
    ncols = nc_ref[...]
    k_lane = lax.broadcasted_iota(jnp.int32, (R, KNN), 1)

    def pick2(k, acc, last=False):
        v = v_ref[...]
        m1 = jnp.max(v, axis=1, keepdims=True)
        sel1 = v == m1
        idx1f = jnp.max(jnp.where(sel1, ncols, NEG_INF), axis=1, keepdims=True)
        v2 = jnp.where(sel1, NEG_INF, v)
        m2 = jnp.max(v2, axis=1, keepdims=True)
        sel2 = v2 == m2
        idx2f = jnp.max(jnp.where(sel2, ncols, NEG_INF), axis=1, keepdims=True)
        if not last:
            v_ref[...] = jnp.where(sel2, NEG_INF, v2)
        idx1 = (-idx1f).astype(jnp.int32)
        idx2 = (-idx2f).astype(jnp.int32)
        return (acc + jnp.where(k_lane == 2 * k, idx1, 0)
                + jnp.where(k_lane == 2 * k + 1, idx2, 0))

    acc = lax.fori_loop(0, KNN // 2 - 1, pick2, jnp.zeros((R, KNN), jnp.int32))
    acc = pick2(KNN // 2 - 1, acc, last=True)
    idx_ref[...] = acc + base


def _knn_topk_batch(x, ncols, b, tile_rows=256):
    B, N, D = x.shape
    return pl.pallas_call(
        functools.partial(_knn_body, b * N),
        grid=(N // tile_rows,),
        in_specs=[
            pl.BlockSpec((1, tile_rows, D), lambda i, _b=b: (_b, i, 0)),
            pl.BlockSpec((1, N, D), lambda i, _b=b: (_b, 0, 0)),
            pl.BlockSpec((1, N), lambda i: (0, 0)),
        ],
        out_specs=pl.BlockSpec((tile_rows, KNN), lambda i: (i, 0)),
        out_shape=jax.ShapeDtypeStruct((N, KNN), jnp.int32),
        scratch_shapes=[pltpu.VMEM((tile_rows, N), jnp.float32)],
    )(x, x, ncols)


def _yz_body(x_ref, w_ref, yz_ref):
    D = x_ref.shape[1]
    w1 = w_ref[:, :D]
    w_cat = jnp.concatenate([w1, w_ref[:, D:] - w1], axis=0)
    yz_ref[...] = lax.dot_general(
        x_ref[...], w_cat, (((1,), (1,)), ((), ())),
        preferred_element_type=jnp.float32, precision=lax.Precision.HIGHEST)


def _yz_matmul(x_flat, w, tile_rows=1024):
    BN, D = x_flat.shape
    OC = w.shape[0]
    return pl.pallas_call(
        _yz_body,
        grid=(BN // tile_rows,),
        in_specs=[
            pl.BlockSpec((tile_rows, D), lambda i: (i, 0)),
            pl.BlockSpec((OC, 2 * D), lambda i: (0, 0)),
        ],
        out_specs=pl.BlockSpec((tile_rows, 2 * OC), lambda i: (i, 0)),
        out_shape=jax.ShapeDtypeStruct((BN, 2 * OC), jnp.float32),
    )(x_flat, w)


def _sc_gather_reduce(y_flat, idx_flat):
    _, OC = y_flat.shape
    P = idx_flat.shape[0] // KNN
    NW = 32
    rows_per_w = P // NW
    G = 4
    GK = G * KNN
    groups = rows_per_w // G
    mesh = plsc.VectorSubcoreMesh(core_axis_name="c", subcore_axis_name="s")

    @functools.partial(
        pl.kernel, mesh=mesh,
        out_type=[
            jax.ShapeDtypeStruct((P, OC), jnp.float32),
            jax.ShapeDtypeStruct((P, OC), jnp.float32),
            jax.ShapeDtypeStruct((NW, OC), jnp.float32),
        ],
        scratch_types=[
            pltpu.VMEM((2, GK), jnp.int32),
            pltpu.VMEM((2, GK, 256), jnp.float32),
            pltpu.VMEM((2, G, 256), jnp.float32),
            pltpu.VMEM((2, G, 256), jnp.float32),
            pltpu.VMEM((1, 256), jnp.float32),
            pltpu.SemaphoreType.DMA,
            pltpu.SemaphoreType.DMA,
            pltpu.SemaphoreType.DMA,
            pltpu.SemaphoreType.DMA,
            pltpu.SemaphoreType.DMA,
            pltpu.SemaphoreType.DMA,
        ],
    )
    def sc_kernel(y_hbm, idx_hbm, m_hbm, s_hbm, t2_hbm,
                  idx_v, rows_v, m_v, s_v, t2_v,
                  si0, si1, sr0, sr1, sw0, sw1):
        sem_i = (si0, si1)
        sem_r = (sr0, sr1)
        sem_w = (sw0, sw1)
        wid = lax.axis_index("s") * 2 + lax.axis_index("c")
        base_n = wid * rows_per_w
        for c in range(OC // 16):
            t2_v[0, pl.ds(c * 16, 16)] = jnp.zeros((16,), jnp.float32)

        def idx_copy(g, b):
            return pltpu.make_async_copy(
                idx_hbm.at[pl.ds((base_n + g * G) * KNN, GK)],
                idx_v.at[b], sem_i[b])

        def rows_copy(b):
            return pltpu.make_async_copy(y_hbm.at[idx_v.at[b]],
                                         rows_v.at[b], sem_r[b])

        def write_copies(g, b):
            row0 = base_n + g * G
            return (
                pltpu.make_async_copy(m_v.at[b], m_hbm.at[pl.ds(row0, G)],
                                      sem_w[b]),
                pltpu.make_async_copy(s_v.at[b], s_hbm.at[pl.ds(row0, G)],
                                      sem_w[b]),
            )

        def compute(b):
            def per_point(j, carry2):
                for c in range(OC // 16):
                    sl = pl.ds(c * 16, 16)
                    v0 = rows_v[b, j * KNN, sl]
                    acc_m = v0
                    acc_s = v0
                    acc_q = v0 * v0
                    for kk in range(1, KNN):
                        v = rows_v[b, j * KNN + kk, sl]
                        acc_m = jnp.maximum(acc_m, v)
                        acc_s = acc_s + v
                        acc_q = acc_q + v * v
                    m_v[b, j, sl] = acc_m
                    s_v[b, j, sl] = acc_s
                    t2_v[0, sl] = t2_v[0, sl] + acc_q
                return carry2

            lax.fori_loop(0, G, per_point, 0)

        idx_copy(0, 0).start()
        idx_copy(0, 0).wait()
        rows_copy(0).start()
        idx_copy(1, 1).start()

        def steady(g, b):
            rows_copy(b).wait()
            idx_copy(g + 1, 1 - b).wait()
            rows_copy(1 - b).start()
            idx_copy(g + 2, b).start()

            @pl.when(g >= 2)
            def _drain():
                wm, ws = write_copies(g - 2, b)
                wm.wait()
                ws.wait()

            compute(b)
            wm, ws = write_copies(g, b)
            wm.start()
            ws.start()

        def pair(p, carry):
            steady(2 * p, 0)
            steady(2 * p + 1, 1)
            return carry

        lax.fori_loop(0, groups // 2 - 1, pair, 0)

        g0 = groups - 2
        rows_copy(0).wait()
        idx_copy(g0 + 1, 1).wait()
        rows_copy(1).start()
        wm, ws = write_copies(g0 - 2, 0)
        wm.wait()
        ws.wait()
        compute(0)
        wm, ws = write_copies(g0, 0)
        wm.start()
        ws.start()

        g1 = groups - 1
        rows_copy(1).wait()
        wm, ws = write_copies(g1 - 2, 1)
        wm.wait()
        ws.wait()
        compute(1)
        wm, ws = write_copies(g1, 1)
        wm.start()
        ws.start()

        wm, ws = write_copies(g0, 0)
        wm.wait()
        ws.wait()
        wm, ws = write_copies(g1, 1)
        wm.wait()
        ws.wait()
        pltpu.sync_copy(t2_v, t2_hbm.at[pl.ds(wid, 1)])

    return sc_kernel(y_flat, idx_flat)


def _stats_body(s_ref, z_ref, t2p_ref, t1_ref, z1_ref, z2_ref, zs_ref, t2_ref):
    i = pl.program_id(0)

    @pl.when(i == 0)
    def _init():
        zero = jnp.zeros_like(t1_ref)
        t1_ref[...] = zero
        z1_ref[...] = zero
        z2_ref[...] = zero
        zs_ref[...] = zero
        t2_ref[...] = jnp.sum(t2p_ref[...], axis=0, keepdims=True)

    s = s_ref[...]
    z = z_ref[...]
    t1_ref[...] += jnp.sum(s, axis=0, keepdims=True)
    z1_ref[...] += jnp.sum(z, axis=0, keepdims=True)
    z2_ref[...] += jnp.sum(z * z, axis=0, keepdims=True)
    zs_ref[...] += jnp.sum(z * s, axis=0, keepdims=True)


def _stats(s_arr, z_arr, t2p, z_row0, tile_rows=1024):
    P, OC = s_arr.shape
    zoff = z_row0 // tile_rows
    vec = jax.ShapeDtypeStruct((1, OC), jnp.float32)
    vspec = pl.BlockSpec((1, OC), lambda i: (0, 0))
    return pl.pallas_call(
        _stats_body,
        grid=(P // tile_rows,),
        in_specs=[
            pl.BlockSpec((tile_rows, OC), lambda i: (i, 0)),
            pl.BlockSpec((tile_rows, OC), lambda i, _o=zoff: (_o + i, 0)),
            pl.BlockSpec(t2p.shape, lambda i: (0, 0)),
        ],
        out_specs=[vspec, vspec, vspec, vspec, vspec],
        out_shape=[vec, vec, vec, vec, vec],
    )(s_arr, z_arr, t2p)


def _final_body(z_ref, m_ref, a_ref, bias_ref, o_ref):
    o_ref[...] = jnp.maximum(
        (z_ref[...] + m_ref[...]) * a_ref[...] + bias_ref[...], 0.0)


def _final(z_arr, m_arr, a_vec, bias_vec, tile_rows=1024):
    BN, OC = z_arr.shape
    return pl.pallas_call(
        _final_body,
        grid=(BN // tile_rows,),
        in_specs=[
            pl.BlockSpec((tile_rows, OC), lambda i: (i, 0)),
            pl.BlockSpec((tile_rows, OC), lambda i: (i, 0)),
            pl.BlockSpec((1, OC), lambda i: (0, 0)),
            pl.BlockSpec((1, OC), lambda i: (0, 0)),
        ],
        out_specs=pl.BlockSpec((tile_rows, OC), lambda i: (i, 0)),
        out_shape=jax.ShapeDtypeStruct((BN, OC), jnp.float32),
    )(z_arr, m_arr, a_vec, bias_vec)


def kernel(x, W, gamma, beta):
    B, N, D = x.shape
    OC = W.shape[0]
    BN = B * N

    yz = _yz_matmul(x.reshape(BN, D), W)
    x, yz = lax.optimization_barrier((x, yz))
    y = yz[:, :OC]
    z = yz[:, OC:]

    ncols = -lax.iota(jnp.float32, N)[None, :]
    m_parts, stat_parts = [], []
    for b in range(B):
        idx_b = _knn_topk_batch(x, ncols, b)
        m_b, s_b, t2p_b = _sc_gather_reduce(y, idx_b.reshape(N * KNN))
        m_parts.append(m_b)
        stat_parts.append(_stats(s_b, z, t2p_b, b * N))
    m_arr = jnp.concatenate(m_parts, axis=0)

    t1, z1, z2, zs, t2 = (sum(p[i] for p in stat_parts) for i in range(5))
    cnt = float(BN * KNN)
    mean = (t1 + KNN * z1) / cnt
    e2 = (t2 + 2.0 * zs + KNN * z2) / cnt
    var = e2 - mean * mean
    inv = lax.rsqrt(var + 1e-5)
    a_vec = gamma[None, :] * inv
    bias_vec = beta[None, :] - mean * a_vec

    out = _final(z, m_arr, a_vec, bias_vec)
    return out.reshape(B, N, OC)


# speedup vs baseline: 9.7634x; 1.1334x over previous
import functools

import jax
import jax.numpy as jnp
from jax import lax
from jax.experimental import pallas as pl
from jax.experimental.pallas import tpu as pltpu
from jax.experimental.pallas import tpu_sc as plsc

KNN = 20
NEG_INF = float("-inf")


def _knn_body(base, xr_ref, xf_ref, nc_ref, idx_ref, v_ref):
    xr = xr_ref[0]
    xf = xf_ref[0]
    dot = lax.dot_general(xr, xf, (((1,), (1,)), ((), ())),
                          preferred_element_type=jnp.float32)
    xx_r = jnp.sum(xr * xr, axis=1, keepdims=True)
    xx_f = jnp.sum(xf * xf, axis=1)[None, :]
    v_ref[...] = (2.0 * dot - xx_r) - xx_f
    R, N = v_ref.shape
    ncols = nc_ref[...]
    k_lane = lax.broadcasted_iota(jnp.int32, (R, KNN), 1)

    def pick2(k, acc, last=False):
        v = v_ref[...]
        m1 = jnp.max(v, axis=1, keepdims=True)
        sel1 = v == m1
        idx1f = jnp.max(jnp.where(sel1, ncols, NEG_INF), axis=1, keepdims=True)
        v2 = jnp.where(sel1, NEG_INF, v)
        m2 = jnp.max(v2, axis=1, keepdims=True)
        sel2 = v2 == m2
        idx2f = jnp.max(jnp.where(sel2, ncols, NEG_INF), axis=1, keepdims=True)
        if not last:
            v_ref[...] = jnp.where(sel2, NEG_INF, v2)
        idx1 = (-idx1f).astype(jnp.int32)
        idx2 = (-idx2f).astype(jnp.int32)
        return (acc + jnp.where(k_lane == 2 * k, idx1, 0)
                + jnp.where(k_lane == 2 * k + 1, idx2, 0))

    acc = lax.fori_loop(0, KNN // 2 - 1, pick2, jnp.zeros((R, KNN), jnp.int32))
    acc = pick2(KNN // 2 - 1, acc, last=True)
    idx_ref[...] = acc + base


def _knn_topk_batch(x, ncols, b, tile_rows=512):
    B, N, D = x.shape
    return pl.pallas_call(
        functools.partial(_knn_body, b * N),
        grid=(N // tile_rows,),
        in_specs=[
            pl.BlockSpec((1, tile_rows, D), lambda i, _b=b: (_b, i, 0)),
            pl.BlockSpec((1, N, D), lambda i, _b=b: (_b, 0, 0)),
            pl.BlockSpec((1, N), lambda i: (0, 0)),
        ],
        out_specs=pl.BlockSpec((tile_rows, KNN), lambda i: (i, 0)),
        out_shape=jax.ShapeDtypeStruct((N, KNN), jnp.int32),
        scratch_shapes=[pltpu.VMEM((tile_rows, N), jnp.float32)],
    )(x, x, ncols)


def _yz_body(x_ref, w_ref, y_ref, z_ref):
    D = x_ref.shape[1]
    w1 = w_ref[:, :D]
    w_cat = jnp.concatenate([w1, w_ref[:, D:] - w1], axis=0)
    yz = lax.dot_general(
        x_ref[...], w_cat, (((1,), (1,)), ((), ())),
        preferred_element_type=jnp.float32, precision=lax.Precision.HIGHEST)
    OC = y_ref.shape[1]
    y_ref[...] = yz[:, :OC]
    z_ref[...] = yz[:, OC:]


def _yz_matmul(x_flat, w, tile_rows=1024):
    BN, D = x_flat.shape
    OC = w.shape[0]
    rspec = pl.BlockSpec((tile_rows, OC), lambda i: (i, 0))
    return pl.pallas_call(
        _yz_body,
        grid=(BN // tile_rows,),
        in_specs=[
            pl.BlockSpec((tile_rows, D), lambda i: (i, 0)),
            pl.BlockSpec((OC, 2 * D), lambda i: (0, 0)),
        ],
        out_specs=[rspec, rspec],
        out_shape=[jax.ShapeDtypeStruct((BN, OC), jnp.float32),
                   jax.ShapeDtypeStruct((BN, OC), jnp.float32)],
    )(x_flat, w)


def _sc_gather_reduce(y_flat, idx_flat):
    _, OC = y_flat.shape
    P = idx_flat.shape[0] // KNN
    NW = 32
    rows_per_w = P // NW
    G = 4
    GK = G * KNN
    groups = rows_per_w // G
    mesh = plsc.VectorSubcoreMesh(core_axis_name="c", subcore_axis_name="s")

    @functools.partial(
        pl.kernel, mesh=mesh,
        out_type=[
            jax.ShapeDtypeStruct((P, OC), jnp.float32),
            jax.ShapeDtypeStruct((P, OC), jnp.float32),
            jax.ShapeDtypeStruct((NW, OC), jnp.float32),
        ],
        scratch_types=[
            pltpu.VMEM((2, GK), jnp.int32),
            pltpu.VMEM((2, GK, 256), jnp.float32),
            pltpu.VMEM((2, G, 256), jnp.float32),
            pltpu.VMEM((2, G, 256), jnp.float32),
            pltpu.VMEM((1, 256), jnp.float32),
            pltpu.SemaphoreType.DMA,
            pltpu.SemaphoreType.DMA,
            pltpu.SemaphoreType.DMA,
            pltpu.SemaphoreType.DMA,
            pltpu.SemaphoreType.DMA,
            pltpu.SemaphoreType.DMA,
        ],
    )
    def sc_kernel(y_hbm, idx_hbm, m_hbm, s_hbm, t2_hbm,
                  idx_v, rows_v, m_v, s_v, t2_v,
                  si0, si1, sr0, sr1, sw0, sw1):
        sem_i = (si0, si1)
        sem_r = (sr0, sr1)
        sem_w = (sw0, sw1)
        wid = lax.axis_index("s") * 2 + lax.axis_index("c")
        base_n = wid * rows_per_w
        for c in range(OC // 16):
            t2_v[0, pl.ds(c * 16, 16)] = jnp.zeros((16,), jnp.float32)

        def idx_copy(g, b):
            return pltpu.make_async_copy(
                idx_hbm.at[pl.ds((base_n + g * G) * KNN, GK)],
                idx_v.at[b], sem_i[b])

        def rows_copy(b):
            return pltpu.make_async_copy(y_hbm.at[idx_v.at[b]],
                                         rows_v.at[b], sem_r[b])

        def write_copies(g, b):
            row0 = base_n + g * G
            return (
                pltpu.make_async_copy(m_v.at[b], m_hbm.at[pl.ds(row0, G)],
                                      sem_w[b]),
                pltpu.make_async_copy(s_v.at[b], s_hbm.at[pl.ds(row0, G)],
                                      sem_w[b]),
            )

        def compute(b):
            def per_point(j, carry2):
                for c in range(OC // 16):
                    sl = pl.ds(c * 16, 16)
                    v0 = rows_v[b, j * KNN, sl]
                    acc_m = v0
                    acc_s = v0
                    acc_q = v0 * v0
                    for kk in range(1, KNN):
                        v = rows_v[b, j * KNN + kk, sl]
                        acc_m = jnp.maximum(acc_m, v)
                        acc_s = acc_s + v
                        acc_q = acc_q + v * v
                    m_v[b, j, sl] = acc_m
                    s_v[b, j, sl] = acc_s
                    t2_v[0, sl] = t2_v[0, sl] + acc_q
                return carry2

            lax.fori_loop(0, G, per_point, 0)

        idx_copy(0, 0).start()
        idx_copy(0, 0).wait()
        rows_copy(0).start()
        idx_copy(1, 1).start()

        def steady(g, b):
            rows_copy(b).wait()
            idx_copy(g + 1, 1 - b).wait()
            rows_copy(1 - b).start()
            idx_copy(g + 2, b).start()

            @pl.when(g >= 2)
            def _drain():
                wm, ws = write_copies(g - 2, b)
                wm.wait()
                ws.wait()

            compute(b)
            wm, ws = write_copies(g, b)
            wm.start()
            ws.start()

        def pair(p, carry):
            steady(2 * p, 0)
            steady(2 * p + 1, 1)
            return carry

        lax.fori_loop(0, groups // 2 - 1, pair, 0)

        g0 = groups - 2
        rows_copy(0).wait()
        idx_copy(g0 + 1, 1).wait()
        rows_copy(1).start()
        wm, ws = write_copies(g0 - 2, 0)
        wm.wait()
        ws.wait()
        compute(0)
        wm, ws = write_copies(g0, 0)
        wm.start()
        ws.start()

        g1 = groups - 1
        rows_copy(1).wait()
        wm, ws = write_copies(g1 - 2, 1)
        wm.wait()
        ws.wait()
        compute(1)
        wm, ws = write_copies(g1, 1)
        wm.start()
        ws.start()

        wm, ws = write_copies(g0, 0)
        wm.wait()
        ws.wait()
        wm, ws = write_copies(g1, 1)
        wm.wait()
        ws.wait()
        pltpu.sync_copy(t2_v, t2_hbm.at[pl.ds(wid, 1)])

    return sc_kernel(y_flat, idx_flat)


def _stats_body(s_ref, z_ref, t2p_ref, t1_ref, z1_ref, z2_ref, zs_ref, t2_ref):
    i = pl.program_id(0)

    @pl.when(i == 0)
    def _init():
        zero = jnp.zeros_like(t1_ref)
        t1_ref[...] = zero
        z1_ref[...] = zero
        z2_ref[...] = zero
        zs_ref[...] = zero
        t2_ref[...] = jnp.sum(t2p_ref[...], axis=0, keepdims=True)

    s = s_ref[...]
    z = z_ref[...]
    t1_ref[...] += jnp.sum(s, axis=0, keepdims=True)
    z1_ref[...] += jnp.sum(z, axis=0, keepdims=True)
    z2_ref[...] += jnp.sum(z * z, axis=0, keepdims=True)
    zs_ref[...] += jnp.sum(z * s, axis=0, keepdims=True)


def _stats(s_arr, z_arr, t2p, z_row0, tile_rows=1024):
    P, OC = s_arr.shape
    zoff = z_row0 // tile_rows
    vec = jax.ShapeDtypeStruct((1, OC), jnp.float32)
    vspec = pl.BlockSpec((1, OC), lambda i: (0, 0))
    return pl.pallas_call(
        _stats_body,
        grid=(P // tile_rows,),
        in_specs=[
            pl.BlockSpec((tile_rows, OC), lambda i: (i, 0)),
            pl.BlockSpec((tile_rows, OC), lambda i, _o=zoff: (_o + i, 0)),
            pl.BlockSpec(t2p.shape, lambda i: (0, 0)),
        ],
        out_specs=[vspec, vspec, vspec, vspec, vspec],
        out_shape=[vec, vec, vec, vec, vec],
    )(s_arr, z_arr, t2p)


def _final_body(z_ref, m_ref, a_ref, bias_ref, o_ref):
    o_ref[...] = jnp.maximum(
        (z_ref[...] + m_ref[...]) * a_ref[...] + bias_ref[...], 0.0)


def _final(z_arr, m_arr, a_vec, bias_vec, tile_rows=1024):
    BN, OC = z_arr.shape
    return pl.pallas_call(
        _final_body,
        grid=(BN // tile_rows,),
        in_specs=[
            pl.BlockSpec((tile_rows, OC), lambda i: (i, 0)),
            pl.BlockSpec((tile_rows, OC), lambda i: (i, 0)),
            pl.BlockSpec((1, OC), lambda i: (0, 0)),
            pl.BlockSpec((1, OC), lambda i: (0, 0)),
        ],
        out_specs=pl.BlockSpec((tile_rows, OC), lambda i: (i, 0)),
        out_shape=jax.ShapeDtypeStruct((BN, OC), jnp.float32),
    )(z_arr, m_arr, a_vec, bias_vec)


def kernel(x, W, gamma, beta):
    B, N, D = x.shape
    OC = W.shape[0]
    BN = B * N

    y, z = _yz_matmul(x.reshape(BN, D), W)
    x, y, z = lax.optimization_barrier((x, y, z))

    ncols = -lax.iota(jnp.float32, N)[None, :]
    m_parts, stat_parts = [], []
    for b in range(B):
        idx_b = _knn_topk_batch(x, ncols, b)
        m_b, s_b, t2p_b = _sc_gather_reduce(y, idx_b.reshape(N * KNN))
        m_parts.append(m_b)
        stat_parts.append(_stats(s_b, z, t2p_b, b * N))
    m_arr = jnp.concatenate(m_parts, axis=0)

    t1, z1, z2, zs, t2 = (sum(p[i] for p in stat_parts) for i in range(5))
    cnt = float(BN * KNN)
    mean = (t1 + KNN * z1) / cnt
    e2 = (t2 + 2.0 * zs + KNN * z2) / cnt
    var = e2 - mean * mean
    inv = lax.rsqrt(var + 1e-5)
    a_vec = gamma[None, :] * inv
    bias_vec = beta[None, :] - mean * a_vec

    out = _final(z, m_arr, a_vec, bias_vec)
    return out.reshape(B, N, OC)
